# baseline (speedup 1.0000x reference)
import functools

import jax
import jax.numpy as jnp
from jax.experimental import pallas as pl
from jax.experimental.pallas import tpu as pltpu


def _round_up(x: int, m: int) -> int:
    return ((x + m - 1) // m) * m


_BB = 8


def _fused_gcn_kernel(text_ref, adj_a, adj_b, w_ref, b_ref, out_ref, *, bb, n):
    f_in = w_ref.shape[1]
    hb = bb // 2
    x = text_ref[...].reshape(bb * n, f_in)
    h = jax.lax.dot_general(x, w_ref[...], (((1,), (1,)), ((), ())),
                            preferred_element_type=jnp.float32)
    h = h + b_ref[...]
    for half, adj_ref in enumerate((adj_a, adj_b)):
        for i in range(hb):
            g = half * hb + i
            adj = adj_ref[i]
            agg = jnp.dot(adj, h[g * n:(g + 1) * n],
                          preferred_element_type=jnp.float32)
            denom = jnp.sum(adj, axis=1, keepdims=True) + 1.0
            inv = pl.reciprocal(denom, approx=False)
            out_ref[g] = (agg * inv).astype(out_ref.dtype)


def kernel(text, adj, weight, bias):
    B, N, F_in = text.shape
    F_out = weight.shape[0]

    N_pad = _round_up(N, 128)
    F_in_pad = _round_up(F_in, 128)
    F_out_pad = _round_up(F_out, 128)
    bb = _BB if B % _BB == 0 else 1
    B_pad = _round_up(B, bb)
    hb = max(bb // 2, 1)

    f32 = jnp.float32
    text_p = jnp.pad(text.astype(f32),
                     ((0, B_pad - B), (0, N_pad - N), (0, F_in_pad - F_in)))
    adj_p = jnp.pad(adj.astype(f32),
                    ((0, B_pad - B), (0, N_pad - N), (0, N_pad - N)))
    w_p = jnp.pad(weight.astype(f32),
                  ((0, F_out_pad - F_out), (0, F_in_pad - F_in)))
    b_p = jnp.pad(bias.astype(f32), (0, F_out_pad - F_out)).reshape(1, -1)

    if bb == 1:
        def body_simple(text_ref, adj_ref, w_ref, b_ref, out_ref):
            x = text_ref[...].reshape(N_pad, F_in_pad)
            h = jax.lax.dot_general(x, w_ref[...], (((1,), (1,)), ((), ())),
                                    preferred_element_type=jnp.float32)
            h = h + b_ref[...]
            adj_m = adj_ref[0]
            agg = jnp.dot(adj_m, h, preferred_element_type=jnp.float32)
            denom = jnp.sum(adj_m, axis=1, keepdims=True) + 1.0
            out_ref[0] = (agg * pl.reciprocal(denom, approx=False)
                          ).astype(out_ref.dtype)

        out_p = pl.pallas_call(
            body_simple,
            out_shape=jax.ShapeDtypeStruct((B_pad, N_pad, F_out_pad),
                                           text.dtype),
            grid=(B_pad,),
            in_specs=[
                pl.BlockSpec((1, N_pad, F_in_pad), lambda i: (i, 0, 0)),
                pl.BlockSpec((1, N_pad, N_pad), lambda i: (i, 0, 0)),
                pl.BlockSpec((F_out_pad, F_in_pad), lambda i: (0, 0)),
                pl.BlockSpec((1, F_out_pad), lambda i: (0, 0)),
            ],
            out_specs=pl.BlockSpec((1, N_pad, F_out_pad), lambda i: (i, 0, 0)),
            compiler_params=pltpu.CompilerParams(
                dimension_semantics=("parallel",)),
        )(text_p, adj_p, w_p, b_p)
        return out_p[:B, :N, :F_out]

    body = functools.partial(_fused_gcn_kernel, bb=bb, n=N_pad)
    out_p = pl.pallas_call(
        body,
        out_shape=jax.ShapeDtypeStruct((B_pad, N_pad, F_out_pad), text.dtype),
        grid=(B_pad // bb,),
        in_specs=[
            pl.BlockSpec((bb, N_pad, F_in_pad), lambda i: (i, 0, 0)),
            pl.BlockSpec((hb, N_pad, N_pad), lambda i: (2 * i, 0, 0)),
            pl.BlockSpec((hb, N_pad, N_pad), lambda i: (2 * i + 1, 0, 0)),
            pl.BlockSpec((F_out_pad, F_in_pad), lambda i: (0, 0)),
            pl.BlockSpec((1, F_out_pad), lambda i: (0, 0)),
        ],
        out_specs=pl.BlockSpec((bb, N_pad, F_out_pad), lambda i: (i, 0, 0)),
        compiler_params=pltpu.CompilerParams(
            dimension_semantics=("parallel",)),
    )(text_p, adj_p, adj_p, w_p, b_p)

    return out_p[:B, :N, :F_out]
